# baseline (speedup 1.0000x reference)
import functools

import jax
import jax.numpy as jnp
from jax import lax
from jax.experimental import pallas as pl
from jax.experimental.pallas import tpu as pltpu
from jax.experimental.pallas import tpu_sc as plsc

H = 128
TILE = 512
GB = 512
LANES = 128

F32 = jnp.float32


def _ln(h, g, beta):
    mu = jnp.mean(h, axis=-1, keepdims=True)
    d = h - mu
    var = jnp.mean(d * d, axis=-1, keepdims=True)
    return d * lax.rsqrt(var + 1e-5) * g + beta


def _enc_node_body(x_ref, m_ref, s_ref, w1_ref, b1_ref, w2_ref, b2_ref,
                   g_ref, be_ref, out_ref):
    xn = (x_ref[...] - m_ref[...]) / s_ref[...]
    h = jnp.dot(xn, w1_ref[...], preferred_element_type=F32) + b1_ref[...]
    h = jnp.maximum(h, 0.0)
    h = jnp.dot(h, w2_ref[...], preferred_element_type=F32) + b2_ref[...]
    out_ref[...] = _ln(h, g_ref[...], be_ref[...])


def _pq_body(x_ref, w_ref, b_ref, out_ref):
    out_ref[0] = jnp.dot(x_ref[...], w_ref[0], preferred_element_type=F32) + b_ref[0]


BF16 = jnp.bfloat16


def _bdot(a, b):
    return jnp.dot(a.astype(BF16), b.astype(BF16), preferred_element_type=F32)


def _edge_first_body(er_ref, m_ref, s_ref, ew1_ref, eb1_ref, ew2_ref, eb2_ref,
                     eg_ref, ebe_ref, gr_ref, gc_ref, w1e_ref, w2_ref, b2_ref,
                     g_ref, be_ref, out_ref):
    ea = (er_ref[...] - m_ref[...]) / s_ref[...]
    h = jnp.dot(ea, ew1_ref[...], preferred_element_type=F32) + eb1_ref[...]
    h = jnp.maximum(h, 0.0)
    h = _bdot(h, ew2_ref[...]) + eb2_ref[...]
    ea_enc = _ln(h, eg_ref[...], ebe_ref[...])
    h1 = gr_ref[...] + gc_ref[...] + _bdot(ea_enc, w1e_ref[...])
    r = jnp.maximum(h1, 0.0)
    h2 = _bdot(r, w2_ref[...]) + b2_ref[...]
    out_ref[...] = ea_enc + _ln(h2, g_ref[...], be_ref[...])


def _edge_rest_body(ea_ref, gr_ref, gc_ref, w1e_ref, w2_ref, b2_ref,
                    g_ref, be_ref, out_ref):
    ea = ea_ref[...]
    h1 = gr_ref[...] + gc_ref[...] + _bdot(ea, w1e_ref[...])
    r = jnp.maximum(h1, 0.0)
    h2 = _bdot(r, w2_ref[...]) + b2_ref[...]
    out_ref[...] = ea + _ln(h2, g_ref[...], be_ref[...])


def _node_body(x_ref, p0_ref, p1_ref, w1a_ref, w1b_ref, b1_ref, w2_ref,
               b2_ref, g_ref, be_ref, out_ref):
    agg = p0_ref[0] + p1_ref[0]
    h = (jnp.dot(x_ref[...], w1a_ref[...], preferred_element_type=F32)
         + jnp.dot(agg, w1b_ref[...], preferred_element_type=F32) + b1_ref[...])
    h = jnp.maximum(h, 0.0)
    h = jnp.dot(h, w2_ref[...], preferred_element_type=F32) + b2_ref[...]
    out_ref[...] = x_ref[...] + _ln(h, g_ref[...], be_ref[...])


def _dec_body(x_ref, w1_ref, b1_ref, w2_ref, b2_ref, out_ref):
    h = jnp.dot(x_ref[...], w1_ref[...], preferred_element_type=F32) + b1_ref[...]
    h = jnp.maximum(h, 0.0)
    out_ref[...] = jnp.dot(h, w2_ref[...], preferred_element_type=F32) + b2_ref[...]


def _full(a):
    nd = a.ndim
    return pl.BlockSpec(a.shape, lambda i, _nd=nd: (0,) * _nd)


def _pad_idx(idx, k):
    ng = idx.shape[0] // (k * LANES)
    idx3 = idx.reshape(ng, k, LANES)
    idx3 = jnp.pad(idx3, ((0, 0), (0, 8 - k), (0, 0)))
    return idx3.reshape(ng * 8, LANES)


@functools.lru_cache(maxsize=None)
def _make_gather(n_rows, n_tab):
    mesh = plsc.VectorSubcoreMesh(core_axis_name="c", subcore_axis_name="s")
    nw = mesh.num_cores * mesh.num_subcores
    ng = n_rows // GB
    k = GB // LANES

    @functools.partial(
        pl.kernel,
        out_type=jax.ShapeDtypeStruct((n_rows, H), F32),
        mesh=mesh,
        scratch_types=[
            pltpu.VMEM((8, LANES), jnp.int32),
            pltpu.VMEM((GB, H), F32),
            pltpu.SemaphoreType.DMA,
        ],
    )
    def gather_k(table, idx2d, out, idx_v, rows_v, sem):
        c = lax.axis_index("c")
        s = lax.axis_index("s")
        w = s * mesh.num_cores + c
        cnt = (ng - w + nw - 1) // nw

        @pl.loop(0, cnt)
        def _(t):
            g = w + t * nw
            pltpu.sync_copy(idx2d.at[pl.ds(g * 8, 8)], idx_v)
            descs = [
                pltpu.async_copy(table.at[idx_v.at[j]],
                                 rows_v.at[pl.ds(j * LANES, LANES)], sem)
                for j in range(k)
            ]
            for d in descs:
                d.wait()
            pltpu.sync_copy(rows_v, out.at[pl.ds(g * GB, GB)])

    return gather_k


@functools.lru_cache(maxsize=None)
def _make_scatter(n_edges, n_nodes):
    mesh = plsc.VectorSubcoreMesh(core_axis_name="c", subcore_axis_name="s")
    nc, ns = mesh.num_cores, mesh.num_subcores
    nw = nc * ns
    gb = 256
    ng = n_edges // gb
    k = gb // LANES
    assert n_nodes % 8 == 0
    nz = (n_nodes // ns) & ~7
    tail = n_nodes - ns * nz
    assert tail % 8 == 0 and tail <= gb
    zfull, zrem = nz // gb, nz % gb

    @functools.partial(
        pl.kernel,
        out_type=jax.ShapeDtypeStruct((nc, n_nodes, H), F32),
        mesh=mesh,
        scratch_types=[
            pltpu.VMEM((8, LANES), jnp.int32),
            pltpu.VMEM((gb, H), F32),
            pltpu.VMEM_SHARED((n_nodes, H), F32),
            pltpu.SemaphoreType.DMA,
        ],
    )
    def scatter_k(ea, col2d, out, idx_v, rows_v, acc, sem):
        c = lax.axis_index("c")
        s = lax.axis_index("s")
        w = s * nc + c
        base = s * nz

        @pl.loop(0, gb)
        def _(i):
            for j in range(H // 16):
                rows_v[i, pl.ds(j * 16, 16)] = jnp.zeros((16,), F32)

        for q in range(zfull):
            pltpu.sync_copy(rows_v, acc.at[pl.ds(base + q * gb, gb)])
        if zrem:
            pltpu.sync_copy(rows_v.at[pl.ds(0, zrem)],
                            acc.at[pl.ds(base + zfull * gb, zrem)])
        if tail:
            @pl.when(s == 0)
            def _():
                pltpu.sync_copy(rows_v.at[pl.ds(0, tail)],
                                acc.at[pl.ds(ns * nz, tail)])
        plsc.subcore_barrier()

        cnt = (ng - w + nw - 1) // nw

        @pl.loop(0, cnt)
        def _(t):
            g = w + t * nw
            pltpu.sync_copy(col2d.at[pl.ds(g * 8, 8)], idx_v)
            pltpu.sync_copy(ea.at[pl.ds(g * gb, gb)], rows_v)
            for j in range(k):
                pltpu.sync_copy(rows_v.at[pl.ds(j * LANES, LANES)],
                                acc.at[idx_v.at[j]], add=True)

        plsc.subcore_barrier()
        pltpu.sync_copy(acc.at[pl.ds(base, nz)], out.at[c, pl.ds(base, nz)])
        if tail:
            @pl.when(s == 0)
            def _():
                pltpu.sync_copy(acc.at[pl.ds(ns * nz, tail)],
                                out.at[c, pl.ds(ns * nz, tail)])

    return scatter_k


def kernel(x, edge_index, edge_attr, mean_vec_x, std_vec_x, mean_vec_edge,
           std_vec_edge, params):
    n, dn = x.shape
    e, de = edge_attr.shape
    assert e % GB == 0 and e % LANES == 0
    nblk = -(-n // TILE)
    npad = nblk * TILE
    eblk = e // TILE

    row = edge_index[0]
    col = edge_index[1]
    idx_all = _pad_idx(jnp.concatenate([row, col + npad]), GB // LANES)
    col2d = _pad_idx(col, 256 // LANES)

    r1 = lambda v: v.reshape(1, -1)
    par = pltpu.CompilerParams(dimension_semantics=("parallel",))
    par2 = pltpu.CompilerParams(dimension_semantics=("parallel", "parallel"))

    pe = params["node_enc"]
    x_enc = pl.pallas_call(
        _enc_node_body,
        grid=(nblk,),
        in_specs=[pl.BlockSpec((TILE, dn), lambda i: (i, 0))]
        + [_full(r1(mean_vec_x)), _full(r1(std_vec_x)), _full(pe["W1"]),
           _full(r1(pe["b1"])), _full(pe["W2"]), _full(r1(pe["b2"])),
           _full(r1(pe["g"])), _full(r1(pe["beta"]))],
        out_specs=pl.BlockSpec((TILE, H), lambda i: (i, 0)),
        out_shape=jax.ShapeDtypeStruct((n, H), F32),
        compiler_params=par,
    )(x, r1(mean_vec_x), r1(std_vec_x), pe["W1"], r1(pe["b1"]), pe["W2"],
      r1(pe["b2"]), r1(pe["g"]), r1(pe["beta"]))

    def _pq(xh, w1, b1):
        wpq = jnp.stack([w1[:H], w1[H:2 * H]])
        bpq = jnp.stack([jnp.zeros_like(b1), b1]).reshape(2, 1, H)
        return pl.pallas_call(
            _pq_body,
            grid=(2, nblk),
            in_specs=[pl.BlockSpec((TILE, H), lambda j, i: (i, 0)),
                      pl.BlockSpec((1, H, H), lambda j, i: (j, 0, 0)),
                      pl.BlockSpec((1, 1, H), lambda j, i: (j, 0, 0))],
            out_specs=pl.BlockSpec((1, TILE, H), lambda j, i: (j, i, 0)),
            out_shape=jax.ShapeDtypeStruct((2, npad, H), F32),
            compiler_params=par2,
        )(xh, wpq, bpq)

    gather = _make_gather(2 * e, 2 * npad)
    scatter = _make_scatter(e, n)

    ew = params["edge_enc"]
    xh = x_enc
    ea = None
    for li, lp in enumerate(params["layers"]):
        le, ln_ = lp["edge"], lp["node"]
        t = _pq(xh, le["W1"], le["b1"])
        s2 = gather(t.reshape(2 * npad, H), idx_all)

        espec = pl.BlockSpec((TILE, H), lambda i: (i, 0))
        gr_spec = pl.BlockSpec((TILE, H), lambda i: (i, 0))
        gc_spec = pl.BlockSpec((TILE, H), lambda i, _o=eblk: (i + _o, 0))
        w1e = le["W1"][2 * H:]
        if li == 0:
            ea = pl.pallas_call(
                _edge_first_body,
                grid=(eblk,),
                in_specs=[pl.BlockSpec((TILE, de), lambda i: (i, 0)),
                          _full(r1(mean_vec_edge)), _full(r1(std_vec_edge)),
                          _full(ew["W1"]), _full(r1(ew["b1"])), _full(ew["W2"]),
                          _full(r1(ew["b2"])), _full(r1(ew["g"])),
                          _full(r1(ew["beta"])), gr_spec, gc_spec,
                          _full(w1e), _full(le["W2"]), _full(r1(le["b2"])),
                          _full(r1(le["g"])), _full(r1(le["beta"]))],
                out_specs=pl.BlockSpec((TILE, H), lambda i: (i, 0)),
                out_shape=jax.ShapeDtypeStruct((e, H), F32),
                compiler_params=par,
            )(edge_attr, r1(mean_vec_edge), r1(std_vec_edge), ew["W1"],
              r1(ew["b1"]), ew["W2"], r1(ew["b2"]), r1(ew["g"]), r1(ew["beta"]),
              s2, s2, w1e, le["W2"], r1(le["b2"]), r1(le["g"]), r1(le["beta"]))
        else:
            ea = pl.pallas_call(
                _edge_rest_body,
                grid=(eblk,),
                in_specs=[espec, gr_spec, gc_spec, _full(w1e), _full(le["W2"]),
                          _full(r1(le["b2"])), _full(r1(le["g"])),
                          _full(r1(le["beta"]))],
                out_specs=pl.BlockSpec((TILE, H), lambda i: (i, 0)),
                out_shape=jax.ShapeDtypeStruct((e, H), F32),
                compiler_params=par,
            )(ea, s2, s2, w1e, le["W2"], r1(le["b2"]), r1(le["g"]),
              r1(le["beta"]))

        partials = scatter(ea, col2d)

        xh = pl.pallas_call(
            _node_body,
            grid=(nblk,),
            in_specs=[pl.BlockSpec((TILE, H), lambda i: (i, 0)),
                      pl.BlockSpec((1, TILE, H), lambda i: (0, i, 0)),
                      pl.BlockSpec((1, TILE, H), lambda i: (1, i, 0)),
                      _full(ln_["W1"][:H]), _full(ln_["W1"][H:]),
                      _full(r1(ln_["b1"])), _full(ln_["W2"]),
                      _full(r1(ln_["b2"])), _full(r1(ln_["g"])),
                      _full(r1(ln_["beta"]))],
            out_specs=pl.BlockSpec((TILE, H), lambda i: (i, 0)),
            out_shape=jax.ShapeDtypeStruct((n, H), F32),
            compiler_params=par,
        )(xh, partials, partials, ln_["W1"][:H], ln_["W1"][H:], r1(ln_["b1"]),
          ln_["W2"], r1(ln_["b2"]), r1(ln_["g"]), r1(ln_["beta"]))

    do = params["dec_W2"].shape[1]
    w2p = jnp.zeros((H, H), F32).at[:, :do].set(params["dec_W2"])
    b2p = jnp.zeros((1, H), F32).at[0, :do].set(params["dec_b2"])
    out = pl.pallas_call(
        _dec_body,
        grid=(nblk,),
        in_specs=[pl.BlockSpec((TILE, H), lambda i: (i, 0)),
                  _full(params["dec_W1"]), _full(r1(params["dec_b1"])),
                  _full(w2p), _full(b2p)],
        out_specs=pl.BlockSpec((TILE, H), lambda i: (i, 0)),
        out_shape=jax.ShapeDtypeStruct((n, H), F32),
        compiler_params=par,
    )(xh, params["dec_W1"], r1(params["dec_b1"]), w2p, b2p)
    return out[:, :do]


# speedup vs baseline: 1.0111x; 1.0111x over previous
import functools

import jax
import jax.numpy as jnp
from jax import lax
from jax.experimental import pallas as pl
from jax.experimental.pallas import tpu as pltpu
from jax.experimental.pallas import tpu_sc as plsc

H = 128
TILE = 512
GB = 512
LANES = 128

F32 = jnp.float32


def _ln(h, g, beta):
    mu = jnp.mean(h, axis=-1, keepdims=True)
    d = h - mu
    var = jnp.mean(d * d, axis=-1, keepdims=True)
    return d * lax.rsqrt(var + 1e-5) * g + beta


def _enc_node_body(x_ref, m_ref, s_ref, w1_ref, b1_ref, w2_ref, b2_ref,
                   g_ref, be_ref, out_ref):
    xn = (x_ref[...] - m_ref[...]) / s_ref[...]
    h = jnp.dot(xn, w1_ref[...], preferred_element_type=F32) + b1_ref[...]
    h = jnp.maximum(h, 0.0)
    h = jnp.dot(h, w2_ref[...], preferred_element_type=F32) + b2_ref[...]
    out_ref[...] = _ln(h, g_ref[...], be_ref[...])


def _pq_body(x_ref, w_ref, b_ref, out_ref):
    out_ref[0] = jnp.dot(x_ref[...], w_ref[0], preferred_element_type=F32) + b_ref[0]


BF16 = jnp.bfloat16


def _bdot(a, b):
    return jnp.dot(a.astype(BF16), b.astype(BF16), preferred_element_type=F32)


def _edge_first_body(er_ref, m_ref, s_ref, ew1_ref, eb1_ref, ew2_ref, eb2_ref,
                     eg_ref, ebe_ref, gr_ref, gc_ref, w1e_ref, w2_ref, b2_ref,
                     g_ref, be_ref, out_ref):
    ea = (er_ref[...] - m_ref[...]) / s_ref[...]
    h = lax.dot_general(ea, ew1_ref[...], (((0,), (0,)), ((), ())),
                        preferred_element_type=F32) + eb1_ref[...]
    h = jnp.maximum(h, 0.0)
    h = _bdot(h, ew2_ref[...]) + eb2_ref[...]
    ea_enc = _ln(h, eg_ref[...], ebe_ref[...])
    h1 = gr_ref[...] + gc_ref[...] + _bdot(ea_enc, w1e_ref[...])
    r = jnp.maximum(h1, 0.0)
    h2 = _bdot(r, w2_ref[...]) + b2_ref[...]
    out_ref[...] = ea_enc + _ln(h2, g_ref[...], be_ref[...])


def _edge_rest_body(ea_ref, gr_ref, gc_ref, w1e_ref, w2_ref, b2_ref,
                    g_ref, be_ref, out_ref):
    ea = ea_ref[...]
    h1 = gr_ref[...] + gc_ref[...] + _bdot(ea, w1e_ref[...])
    r = jnp.maximum(h1, 0.0)
    h2 = _bdot(r, w2_ref[...]) + b2_ref[...]
    out_ref[...] = ea + _ln(h2, g_ref[...], be_ref[...])


def _node_body(x_ref, p0_ref, p1_ref, w1a_ref, w1b_ref, b1_ref, w2_ref,
               b2_ref, g_ref, be_ref, out_ref):
    agg = p0_ref[0] + p1_ref[0]
    h = (jnp.dot(x_ref[...], w1a_ref[...], preferred_element_type=F32)
         + jnp.dot(agg, w1b_ref[...], preferred_element_type=F32) + b1_ref[...])
    h = jnp.maximum(h, 0.0)
    h = jnp.dot(h, w2_ref[...], preferred_element_type=F32) + b2_ref[...]
    out_ref[...] = x_ref[...] + _ln(h, g_ref[...], be_ref[...])


def _dec_body(x_ref, w1_ref, b1_ref, w2_ref, b2_ref, out_ref):
    h = jnp.dot(x_ref[...], w1_ref[...], preferred_element_type=F32) + b1_ref[...]
    h = jnp.maximum(h, 0.0)
    out_ref[...] = jnp.dot(h, w2_ref[...], preferred_element_type=F32) + b2_ref[...]


def _full(a):
    nd = a.ndim
    return pl.BlockSpec(a.shape, lambda i, _nd=nd: (0,) * _nd)


def _pad_idx(idx, k):
    ng = idx.shape[0] // (k * LANES)
    idx3 = idx.reshape(ng, k, LANES)
    idx3 = jnp.pad(idx3, ((0, 0), (0, 8 - k), (0, 0)))
    return idx3.reshape(ng * 8, LANES)


@functools.lru_cache(maxsize=None)
def _make_gather(n_rows, n_tab):
    mesh = plsc.VectorSubcoreMesh(core_axis_name="c", subcore_axis_name="s")
    nw = mesh.num_cores * mesh.num_subcores
    ng = n_rows // GB
    k = GB // LANES

    @functools.partial(
        pl.kernel,
        out_type=jax.ShapeDtypeStruct((n_rows, H), F32),
        mesh=mesh,
        scratch_types=[
            pltpu.VMEM((8, LANES), jnp.int32),
            pltpu.VMEM((GB, H), F32),
            pltpu.SemaphoreType.DMA,
        ],
    )
    def gather_k(table, idx2d, out, idx_v, rows_v, sem):
        c = lax.axis_index("c")
        s = lax.axis_index("s")
        w = s * mesh.num_cores + c
        cnt = (ng - w + nw - 1) // nw

        @pl.loop(0, cnt)
        def _(t):
            g = w + t * nw
            pltpu.sync_copy(idx2d.at[pl.ds(g * 8, 8)], idx_v)
            descs = [
                pltpu.async_copy(table.at[idx_v.at[j]],
                                 rows_v.at[pl.ds(j * LANES, LANES)], sem)
                for j in range(k)
            ]
            for d in descs:
                d.wait()
            pltpu.sync_copy(rows_v, out.at[pl.ds(g * GB, GB)])

    return gather_k


@functools.lru_cache(maxsize=None)
def _make_scatter(n_edges, n_nodes):
    mesh = plsc.VectorSubcoreMesh(core_axis_name="c", subcore_axis_name="s")
    nc, ns = mesh.num_cores, mesh.num_subcores
    nw = nc * ns
    gb = 256
    ng = n_edges // gb
    k = gb // LANES
    assert n_nodes % 8 == 0
    nz = (n_nodes // ns) & ~7
    tail = n_nodes - ns * nz
    assert tail % 8 == 0 and tail <= gb
    zfull, zrem = nz // gb, nz % gb

    @functools.partial(
        pl.kernel,
        out_type=jax.ShapeDtypeStruct((nc, n_nodes, H), F32),
        mesh=mesh,
        scratch_types=[
            pltpu.VMEM((8, LANES), jnp.int32),
            pltpu.VMEM((gb, H), F32),
            pltpu.VMEM_SHARED((n_nodes, H), F32),
            pltpu.SemaphoreType.DMA,
        ],
    )
    def scatter_k(ea, col2d, out, idx_v, rows_v, acc, sem):
        c = lax.axis_index("c")
        s = lax.axis_index("s")
        w = s * nc + c
        base = s * nz

        @pl.loop(0, gb)
        def _(i):
            for j in range(H // 16):
                rows_v[i, pl.ds(j * 16, 16)] = jnp.zeros((16,), F32)

        for q in range(zfull):
            pltpu.sync_copy(rows_v, acc.at[pl.ds(base + q * gb, gb)])
        if zrem:
            pltpu.sync_copy(rows_v.at[pl.ds(0, zrem)],
                            acc.at[pl.ds(base + zfull * gb, zrem)])
        if tail:
            @pl.when(s == 0)
            def _():
                pltpu.sync_copy(rows_v.at[pl.ds(0, tail)],
                                acc.at[pl.ds(ns * nz, tail)])
        plsc.subcore_barrier()

        cnt = (ng - w + nw - 1) // nw

        @pl.loop(0, cnt)
        def _(t):
            g = w + t * nw
            pltpu.sync_copy(col2d.at[pl.ds(g * 8, 8)], idx_v)
            pltpu.sync_copy(ea.at[pl.ds(g * gb, gb)], rows_v)
            for j in range(k):
                pltpu.sync_copy(rows_v.at[pl.ds(j * LANES, LANES)],
                                acc.at[idx_v.at[j]], add=True)

        plsc.subcore_barrier()
        pltpu.sync_copy(acc.at[pl.ds(base, nz)], out.at[c, pl.ds(base, nz)])
        if tail:
            @pl.when(s == 0)
            def _():
                pltpu.sync_copy(acc.at[pl.ds(ns * nz, tail)],
                                out.at[c, pl.ds(ns * nz, tail)])

    return scatter_k


def kernel(x, edge_index, edge_attr, mean_vec_x, std_vec_x, mean_vec_edge,
           std_vec_edge, params):
    n, dn = x.shape
    e, de = edge_attr.shape
    assert e % GB == 0 and e % LANES == 0
    nblk = -(-n // TILE)
    npad = nblk * TILE
    eblk = e // TILE

    row = edge_index[0]
    col = edge_index[1]
    idx_all = _pad_idx(jnp.concatenate([row, col + npad]), GB // LANES)
    col2d = _pad_idx(col, 256 // LANES)

    r1 = lambda v: v.reshape(1, -1)
    par = pltpu.CompilerParams(dimension_semantics=("parallel",))
    par2 = pltpu.CompilerParams(dimension_semantics=("parallel", "parallel"))

    pe = params["node_enc"]
    x_enc = pl.pallas_call(
        _enc_node_body,
        grid=(nblk,),
        in_specs=[pl.BlockSpec((TILE, dn), lambda i: (i, 0))]
        + [_full(r1(mean_vec_x)), _full(r1(std_vec_x)), _full(pe["W1"]),
           _full(r1(pe["b1"])), _full(pe["W2"]), _full(r1(pe["b2"])),
           _full(r1(pe["g"])), _full(r1(pe["beta"]))],
        out_specs=pl.BlockSpec((TILE, H), lambda i: (i, 0)),
        out_shape=jax.ShapeDtypeStruct((n, H), F32),
        compiler_params=par,
    )(x, r1(mean_vec_x), r1(std_vec_x), pe["W1"], r1(pe["b1"]), pe["W2"],
      r1(pe["b2"]), r1(pe["g"]), r1(pe["beta"]))

    def _pq(xh, w1, b1):
        wpq = jnp.stack([w1[:H], w1[H:2 * H]])
        bpq = jnp.stack([jnp.zeros_like(b1), b1]).reshape(2, 1, H)
        return pl.pallas_call(
            _pq_body,
            grid=(2, nblk),
            in_specs=[pl.BlockSpec((TILE, H), lambda j, i: (i, 0)),
                      pl.BlockSpec((1, H, H), lambda j, i: (j, 0, 0)),
                      pl.BlockSpec((1, 1, H), lambda j, i: (j, 0, 0))],
            out_specs=pl.BlockSpec((1, TILE, H), lambda j, i: (j, i, 0)),
            out_shape=jax.ShapeDtypeStruct((2, npad, H), F32),
            compiler_params=par2,
        )(xh, wpq, bpq)

    gather = _make_gather(2 * e, 2 * npad)
    scatter = _make_scatter(e, n)

    ew = params["edge_enc"]
    xh = x_enc
    ea = None
    for li, lp in enumerate(params["layers"]):
        le, ln_ = lp["edge"], lp["node"]
        t = _pq(xh, le["W1"], le["b1"])
        s2 = gather(t.reshape(2 * npad, H), idx_all)

        espec = pl.BlockSpec((TILE, H), lambda i: (i, 0))
        gr_spec = pl.BlockSpec((TILE, H), lambda i: (i, 0))
        gc_spec = pl.BlockSpec((TILE, H), lambda i, _o=eblk: (i + _o, 0))
        w1e = le["W1"][2 * H:]
        if li == 0:
            ea = pl.pallas_call(
                _edge_first_body,
                grid=(eblk,),
                in_specs=[pl.BlockSpec((de, TILE), lambda i: (0, i)),
                          _full(mean_vec_edge.reshape(de, 1)),
                          _full(std_vec_edge.reshape(de, 1)),
                          _full(ew["W1"]), _full(r1(ew["b1"])), _full(ew["W2"]),
                          _full(r1(ew["b2"])), _full(r1(ew["g"])),
                          _full(r1(ew["beta"])), gr_spec, gc_spec,
                          _full(w1e), _full(le["W2"]), _full(r1(le["b2"])),
                          _full(r1(le["g"])), _full(r1(le["beta"]))],
                out_specs=pl.BlockSpec((TILE, H), lambda i: (i, 0)),
                out_shape=jax.ShapeDtypeStruct((e, H), F32),
                compiler_params=par,
            )(edge_attr.T, mean_vec_edge.reshape(de, 1),
              std_vec_edge.reshape(de, 1), ew["W1"],
              r1(ew["b1"]), ew["W2"], r1(ew["b2"]), r1(ew["g"]), r1(ew["beta"]),
              s2, s2, w1e, le["W2"], r1(le["b2"]), r1(le["g"]), r1(le["beta"]))
        else:
            ea = pl.pallas_call(
                _edge_rest_body,
                grid=(eblk,),
                in_specs=[espec, gr_spec, gc_spec, _full(w1e), _full(le["W2"]),
                          _full(r1(le["b2"])), _full(r1(le["g"])),
                          _full(r1(le["beta"]))],
                out_specs=pl.BlockSpec((TILE, H), lambda i: (i, 0)),
                out_shape=jax.ShapeDtypeStruct((e, H), F32),
                compiler_params=par,
            )(ea, s2, s2, w1e, le["W2"], r1(le["b2"]), r1(le["g"]),
              r1(le["beta"]))

        partials = scatter(ea, col2d)

        xh = pl.pallas_call(
            _node_body,
            grid=(nblk,),
            in_specs=[pl.BlockSpec((TILE, H), lambda i: (i, 0)),
                      pl.BlockSpec((1, TILE, H), lambda i: (0, i, 0)),
                      pl.BlockSpec((1, TILE, H), lambda i: (1, i, 0)),
                      _full(ln_["W1"][:H]), _full(ln_["W1"][H:]),
                      _full(r1(ln_["b1"])), _full(ln_["W2"]),
                      _full(r1(ln_["b2"])), _full(r1(ln_["g"])),
                      _full(r1(ln_["beta"]))],
            out_specs=pl.BlockSpec((TILE, H), lambda i: (i, 0)),
            out_shape=jax.ShapeDtypeStruct((n, H), F32),
            compiler_params=par,
        )(xh, partials, partials, ln_["W1"][:H], ln_["W1"][H:], r1(ln_["b1"]),
          ln_["W2"], r1(ln_["b2"]), r1(ln_["g"]), r1(ln_["beta"]))

    do = params["dec_W2"].shape[1]
    w2p = jnp.zeros((H, H), F32).at[:, :do].set(params["dec_W2"])
    b2p = jnp.zeros((1, H), F32).at[0, :do].set(params["dec_b2"])
    out = pl.pallas_call(
        _dec_body,
        grid=(nblk,),
        in_specs=[pl.BlockSpec((TILE, H), lambda i: (i, 0)),
                  _full(params["dec_W1"]), _full(r1(params["dec_b1"])),
                  _full(w2p), _full(b2p)],
        out_specs=pl.BlockSpec((TILE, H), lambda i: (i, 0)),
        out_shape=jax.ShapeDtypeStruct((n, H), F32),
        compiler_params=par,
    )(xh, params["dec_W1"], r1(params["dec_b1"]), w2p, b2p)
    return out[:, :do]


# speedup vs baseline: 1.3433x; 1.3286x over previous
import functools

import jax
import jax.numpy as jnp
from jax import lax
from jax.experimental import pallas as pl
from jax.experimental.pallas import tpu as pltpu
from jax.experimental.pallas import tpu_sc as plsc

H = 128
TILE = 512
TILE_E = 1280
GB = 512
LANES = 128

F32 = jnp.float32


def _ln(h, g, beta):
    mu = jnp.mean(h, axis=-1, keepdims=True)
    d = h - mu
    var = jnp.mean(d * d, axis=-1, keepdims=True)
    return d * lax.rsqrt(var + 1e-5) * g + beta


def _enc_node_body(x_ref, m_ref, s_ref, w1_ref, b1_ref, w2_ref, b2_ref,
                   g_ref, be_ref, out_ref):
    xn = (x_ref[...] - m_ref[...]) / s_ref[...]
    h = jnp.dot(xn, w1_ref[...], preferred_element_type=F32) + b1_ref[...]
    h = jnp.maximum(h, 0.0)
    h = jnp.dot(h, w2_ref[...], preferred_element_type=F32) + b2_ref[...]
    out_ref[...] = _ln(h, g_ref[...], be_ref[...])


def _pq_body(x_ref, w_ref, b_ref, out_ref):
    out_ref[0] = jnp.dot(x_ref[...], w_ref[0], preferred_element_type=F32) + b_ref[0]


BF16 = jnp.bfloat16


def _bdot(a, b):
    return jnp.dot(a.astype(BF16), b.astype(BF16), preferred_element_type=F32)


def _edge_first_body(er_ref, m_ref, s_ref, ew1_ref, eb1_ref, ew2_ref, eb2_ref,
                     eg_ref, ebe_ref, gr_ref, gc_ref, w1e_ref, w2_ref, b2_ref,
                     g_ref, be_ref, out_ref):
    ea = (er_ref[...] - m_ref[...]) / s_ref[...]
    h = lax.dot_general(ea, ew1_ref[...], (((0,), (0,)), ((), ())),
                        preferred_element_type=F32) + eb1_ref[...]
    h = jnp.maximum(h, 0.0)
    h = _bdot(h, ew2_ref[...]) + eb2_ref[...]
    ea_enc = _ln(h, eg_ref[...], ebe_ref[...])
    h1 = gr_ref[...] + gc_ref[...] + _bdot(ea_enc, w1e_ref[...])
    r = jnp.maximum(h1, 0.0)
    h2 = _bdot(r, w2_ref[...]) + b2_ref[...]
    out_ref[...] = ea_enc + _ln(h2, g_ref[...], be_ref[...])


def _edge_rest_body(ea_ref, gr_ref, gc_ref, w1e_ref, w2_ref, b2_ref,
                    g_ref, be_ref, out_ref):
    ea = ea_ref[...]
    h1 = gr_ref[...] + gc_ref[...] + _bdot(ea, w1e_ref[...])
    r = jnp.maximum(h1, 0.0)
    h2 = _bdot(r, w2_ref[...]) + b2_ref[...]
    out_ref[...] = ea + _ln(h2, g_ref[...], be_ref[...])


def _node_body(x_ref, p0_ref, p1_ref, w1a_ref, w1b_ref, b1_ref, w2_ref,
               b2_ref, g_ref, be_ref, out_ref):
    agg = p0_ref[0] + p1_ref[0]
    h = (jnp.dot(x_ref[...], w1a_ref[...], preferred_element_type=F32)
         + jnp.dot(agg, w1b_ref[...], preferred_element_type=F32) + b1_ref[...])
    h = jnp.maximum(h, 0.0)
    h = jnp.dot(h, w2_ref[...], preferred_element_type=F32) + b2_ref[...]
    out_ref[...] = x_ref[...] + _ln(h, g_ref[...], be_ref[...])


def _dec_body(x_ref, w1_ref, b1_ref, w2_ref, b2_ref, out_ref):
    h = jnp.dot(x_ref[...], w1_ref[...], preferred_element_type=F32) + b1_ref[...]
    h = jnp.maximum(h, 0.0)
    out_ref[...] = jnp.dot(h, w2_ref[...], preferred_element_type=F32) + b2_ref[...]


def _full(a):
    nd = a.ndim
    return pl.BlockSpec(a.shape, lambda i, _nd=nd: (0,) * _nd)


def _pad_idx(idx, k):
    ng = idx.shape[0] // (k * LANES)
    idx3 = idx.reshape(ng, k, LANES)
    idx3 = jnp.pad(idx3, ((0, 0), (0, 8 - k), (0, 0)))
    return idx3.reshape(ng * 8, LANES)


@functools.lru_cache(maxsize=None)
def _make_gather(n_rows, n_tab):
    mesh = plsc.VectorSubcoreMesh(core_axis_name="c", subcore_axis_name="s")
    nw = mesh.num_cores * mesh.num_subcores
    ng = n_rows // GB
    k = GB // LANES

    @functools.partial(
        pl.kernel,
        out_type=jax.ShapeDtypeStruct((n_rows, H), F32),
        mesh=mesh,
        scratch_types=[
            pltpu.VMEM((8, LANES), jnp.int32),
            pltpu.VMEM((GB, H), F32),
            pltpu.SemaphoreType.DMA,
        ],
    )
    def gather_k(table, idx2d, out, idx_v, rows_v, sem):
        c = lax.axis_index("c")
        s = lax.axis_index("s")
        w = s * mesh.num_cores + c
        cnt = (ng - w + nw - 1) // nw

        @pl.loop(0, cnt)
        def _(t):
            g = w + t * nw
            pltpu.sync_copy(idx2d.at[pl.ds(g * 8, 8)], idx_v)
            descs = [
                pltpu.async_copy(table.at[idx_v.at[j]],
                                 rows_v.at[pl.ds(j * LANES, LANES)], sem)
                for j in range(k)
            ]
            for d in descs:
                d.wait()
            pltpu.sync_copy(rows_v, out.at[pl.ds(g * GB, GB)])

    return gather_k


@functools.lru_cache(maxsize=None)
def _make_scatter(n_edges, n_nodes):
    mesh = plsc.VectorSubcoreMesh(core_axis_name="c", subcore_axis_name="s")
    nc, ns = mesh.num_cores, mesh.num_subcores
    nw = nc * ns
    gb = 256
    ng = n_edges // gb
    k = gb // LANES
    assert n_nodes % 8 == 0
    nz = (n_nodes // ns) & ~7
    tail = n_nodes - ns * nz
    assert tail % 8 == 0 and tail <= gb
    zfull, zrem = nz // gb, nz % gb

    @functools.partial(
        pl.kernel,
        out_type=jax.ShapeDtypeStruct((nc, n_nodes, H), F32),
        mesh=mesh,
        scratch_types=[
            pltpu.VMEM((8, LANES), jnp.int32),
            pltpu.VMEM((gb, H), F32),
            pltpu.VMEM_SHARED((n_nodes, H), F32),
            pltpu.SemaphoreType.DMA,
        ],
    )
    def scatter_k(ea, col2d, out, idx_v, rows_v, acc, sem):
        c = lax.axis_index("c")
        s = lax.axis_index("s")
        w = s * nc + c
        base = s * nz

        @pl.loop(0, gb)
        def _(i):
            for j in range(H // 16):
                rows_v[i, pl.ds(j * 16, 16)] = jnp.zeros((16,), F32)

        for q in range(zfull):
            pltpu.sync_copy(rows_v, acc.at[pl.ds(base + q * gb, gb)])
        if zrem:
            pltpu.sync_copy(rows_v.at[pl.ds(0, zrem)],
                            acc.at[pl.ds(base + zfull * gb, zrem)])
        if tail:
            @pl.when(s == 0)
            def _():
                pltpu.sync_copy(rows_v.at[pl.ds(0, tail)],
                                acc.at[pl.ds(ns * nz, tail)])
        plsc.subcore_barrier()

        cnt = (ng - w + nw - 1) // nw

        @pl.loop(0, cnt)
        def _(t):
            g = w + t * nw
            pltpu.sync_copy(col2d.at[pl.ds(g * 8, 8)], idx_v)
            pltpu.sync_copy(ea.at[pl.ds(g * gb, gb)], rows_v)
            for j in range(k):
                pltpu.sync_copy(rows_v.at[pl.ds(j * LANES, LANES)],
                                acc.at[idx_v.at[j]], add=True)

        plsc.subcore_barrier()
        pltpu.sync_copy(acc.at[pl.ds(base, nz)], out.at[c, pl.ds(base, nz)])
        if tail:
            @pl.when(s == 0)
            def _():
                pltpu.sync_copy(acc.at[pl.ds(ns * nz, tail)],
                                out.at[c, pl.ds(ns * nz, tail)])

    return scatter_k


def kernel(x, edge_index, edge_attr, mean_vec_x, std_vec_x, mean_vec_edge,
           std_vec_edge, params):
    n, dn = x.shape
    e, de = edge_attr.shape
    assert e % GB == 0 and e % LANES == 0
    nblk = -(-n // TILE)
    npad = nblk * TILE
    eblk = e // TILE_E
    assert e % TILE_E == 0

    row = edge_index[0]
    col = edge_index[1]
    idx_all = _pad_idx(jnp.concatenate([row, col + npad]), GB // LANES)
    col2d = _pad_idx(col, 256 // LANES)

    r1 = lambda v: v.reshape(1, -1)
    par = pltpu.CompilerParams(dimension_semantics=("parallel",))
    par2 = pltpu.CompilerParams(dimension_semantics=("parallel", "parallel"))

    pe = params["node_enc"]
    x_enc = pl.pallas_call(
        _enc_node_body,
        grid=(nblk,),
        in_specs=[pl.BlockSpec((TILE, dn), lambda i: (i, 0))]
        + [_full(r1(mean_vec_x)), _full(r1(std_vec_x)), _full(pe["W1"]),
           _full(r1(pe["b1"])), _full(pe["W2"]), _full(r1(pe["b2"])),
           _full(r1(pe["g"])), _full(r1(pe["beta"]))],
        out_specs=pl.BlockSpec((TILE, H), lambda i: (i, 0)),
        out_shape=jax.ShapeDtypeStruct((n, H), F32),
        compiler_params=par,
    )(x, r1(mean_vec_x), r1(std_vec_x), pe["W1"], r1(pe["b1"]), pe["W2"],
      r1(pe["b2"]), r1(pe["g"]), r1(pe["beta"]))

    def _pq(xh, w1, b1):
        wpq = jnp.stack([w1[:H], w1[H:2 * H]])
        bpq = jnp.stack([jnp.zeros_like(b1), b1]).reshape(2, 1, H)
        return pl.pallas_call(
            _pq_body,
            grid=(2, nblk),
            in_specs=[pl.BlockSpec((TILE, H), lambda j, i: (i, 0)),
                      pl.BlockSpec((1, H, H), lambda j, i: (j, 0, 0)),
                      pl.BlockSpec((1, 1, H), lambda j, i: (j, 0, 0))],
            out_specs=pl.BlockSpec((1, TILE, H), lambda j, i: (j, i, 0)),
            out_shape=jax.ShapeDtypeStruct((2, npad, H), F32),
            compiler_params=par2,
        )(xh, wpq, bpq)

    gather = _make_gather(2 * e, 2 * npad)
    scatter = _make_scatter(e, n)

    ew = params["edge_enc"]
    xh = x_enc
    ea = None
    for li, lp in enumerate(params["layers"]):
        le, ln_ = lp["edge"], lp["node"]
        t = _pq(xh, le["W1"], le["b1"])
        s2 = gather(t.reshape(2 * npad, H), idx_all)

        espec = pl.BlockSpec((TILE_E, H), lambda i: (i, 0))
        gr_spec = pl.BlockSpec((TILE_E, H), lambda i: (i, 0))
        gc_spec = pl.BlockSpec((TILE_E, H), lambda i, _o=eblk: (i + _o, 0))
        w1e = le["W1"][2 * H:]
        if li == 0:
            ea = pl.pallas_call(
                _edge_first_body,
                grid=(eblk,),
                in_specs=[pl.BlockSpec((de, TILE_E), lambda i: (0, i)),
                          _full(mean_vec_edge.reshape(de, 1)),
                          _full(std_vec_edge.reshape(de, 1)),
                          _full(ew["W1"]), _full(r1(ew["b1"])), _full(ew["W2"]),
                          _full(r1(ew["b2"])), _full(r1(ew["g"])),
                          _full(r1(ew["beta"])), gr_spec, gc_spec,
                          _full(w1e), _full(le["W2"]), _full(r1(le["b2"])),
                          _full(r1(le["g"])), _full(r1(le["beta"]))],
                out_specs=pl.BlockSpec((TILE_E, H), lambda i: (i, 0)),
                out_shape=jax.ShapeDtypeStruct((e, H), F32),
                compiler_params=par,
            )(edge_attr.T, mean_vec_edge.reshape(de, 1),
              std_vec_edge.reshape(de, 1), ew["W1"],
              r1(ew["b1"]), ew["W2"], r1(ew["b2"]), r1(ew["g"]), r1(ew["beta"]),
              s2, s2, w1e, le["W2"], r1(le["b2"]), r1(le["g"]), r1(le["beta"]))
        else:
            ea = pl.pallas_call(
                _edge_rest_body,
                grid=(eblk,),
                in_specs=[espec, gr_spec, gc_spec, _full(w1e), _full(le["W2"]),
                          _full(r1(le["b2"])), _full(r1(le["g"])),
                          _full(r1(le["beta"]))],
                out_specs=pl.BlockSpec((TILE_E, H), lambda i: (i, 0)),
                out_shape=jax.ShapeDtypeStruct((e, H), F32),
                compiler_params=par,
            )(ea, s2, s2, w1e, le["W2"], r1(le["b2"]), r1(le["g"]),
              r1(le["beta"]))

        partials = scatter(ea, col2d)

        xh = pl.pallas_call(
            _node_body,
            grid=(nblk,),
            in_specs=[pl.BlockSpec((TILE, H), lambda i: (i, 0)),
                      pl.BlockSpec((1, TILE, H), lambda i: (0, i, 0)),
                      pl.BlockSpec((1, TILE, H), lambda i: (1, i, 0)),
                      _full(ln_["W1"][:H]), _full(ln_["W1"][H:]),
                      _full(r1(ln_["b1"])), _full(ln_["W2"]),
                      _full(r1(ln_["b2"])), _full(r1(ln_["g"])),
                      _full(r1(ln_["beta"]))],
            out_specs=pl.BlockSpec((TILE, H), lambda i: (i, 0)),
            out_shape=jax.ShapeDtypeStruct((n, H), F32),
            compiler_params=par,
        )(xh, partials, partials, ln_["W1"][:H], ln_["W1"][H:], r1(ln_["b1"]),
          ln_["W2"], r1(ln_["b2"]), r1(ln_["g"]), r1(ln_["beta"]))

    do = params["dec_W2"].shape[1]
    w2p = jnp.zeros((H, H), F32).at[:, :do].set(params["dec_W2"])
    b2p = jnp.zeros((1, H), F32).at[0, :do].set(params["dec_b2"])
    out = pl.pallas_call(
        _dec_body,
        grid=(nblk,),
        in_specs=[pl.BlockSpec((TILE, H), lambda i: (i, 0)),
                  _full(params["dec_W1"]), _full(r1(params["dec_b1"])),
                  _full(w2p), _full(b2p)],
        out_specs=pl.BlockSpec((TILE, H), lambda i: (i, 0)),
        out_shape=jax.ShapeDtypeStruct((n, H), F32),
        compiler_params=par,
    )(xh, params["dec_W1"], r1(params["dec_b1"]), w2p, b2p)
    return out[:, :do]


# speedup vs baseline: 1.4948x; 1.1128x over previous
import functools

import jax
import jax.numpy as jnp
from jax import lax
from jax.experimental import pallas as pl
from jax.experimental.pallas import tpu as pltpu
from jax.experimental.pallas import tpu_sc as plsc

H = 128
TILE = 512
TILE_E = 2560
GB = 512
LANES = 128

F32 = jnp.float32


def _ln(h, g, beta):
    mu = jnp.mean(h, axis=-1, keepdims=True)
    d = h - mu
    var = jnp.mean(d * d, axis=-1, keepdims=True)
    return d * lax.rsqrt(var + 1e-5) * g + beta


def _enc_node_body(x_ref, m_ref, s_ref, w1_ref, b1_ref, w2_ref, b2_ref,
                   g_ref, be_ref, out_ref):
    xn = (x_ref[...] - m_ref[...]) / s_ref[...]
    h = jnp.dot(xn, w1_ref[...], preferred_element_type=F32) + b1_ref[...]
    h = jnp.maximum(h, 0.0)
    h = jnp.dot(h, w2_ref[...], preferred_element_type=F32) + b2_ref[...]
    out_ref[...] = _ln(h, g_ref[...], be_ref[...])


def _pq_body(x_ref, w_ref, b_ref, out_ref):
    out_ref[0] = jnp.dot(x_ref[...], w_ref[0], preferred_element_type=F32) + b_ref[0]


BF16 = jnp.bfloat16


def _bdot(a, b):
    return jnp.dot(a.astype(BF16), b.astype(BF16), preferred_element_type=F32)


def _edge_first_body(er_ref, m_ref, s_ref, ew1_ref, eb1_ref, ew2_ref, eb2_ref,
                     eg_ref, ebe_ref, gr_ref, gc_ref, w1e_ref, w2_ref, b2_ref,
                     g_ref, be_ref, out_ref):
    ea = (er_ref[...] - m_ref[...]) / s_ref[...]
    h = lax.dot_general(ea, ew1_ref[...], (((0,), (0,)), ((), ())),
                        preferred_element_type=F32) + eb1_ref[...]
    h = jnp.maximum(h, 0.0)
    h = _bdot(h, ew2_ref[...]) + eb2_ref[...]
    ea_enc = _ln(h, eg_ref[...], ebe_ref[...])
    h1 = gr_ref[...] + gc_ref[...] + _bdot(ea_enc, w1e_ref[...])
    r = jnp.maximum(h1, 0.0)
    h2 = _bdot(r, w2_ref[...]) + b2_ref[...]
    out_ref[...] = ea_enc + _ln(h2, g_ref[...], be_ref[...])


def _edge_rest_body(ea_ref, gr_ref, gc_ref, w1e_ref, w2_ref, b2_ref,
                    g_ref, be_ref, out_ref):
    ea = ea_ref[...]
    h1 = gr_ref[...] + gc_ref[...] + _bdot(ea, w1e_ref[...])
    r = jnp.maximum(h1, 0.0)
    h2 = _bdot(r, w2_ref[...]) + b2_ref[...]
    out_ref[...] = ea + _ln(h2, g_ref[...], be_ref[...])


def _node_body(x_ref, p0_ref, p1_ref, w1a_ref, w1b_ref, b1_ref, w2_ref,
               b2_ref, g_ref, be_ref, out_ref):
    agg = p0_ref[0] + p1_ref[0]
    h = (jnp.dot(x_ref[...], w1a_ref[...], preferred_element_type=F32)
         + jnp.dot(agg, w1b_ref[...], preferred_element_type=F32) + b1_ref[...])
    h = jnp.maximum(h, 0.0)
    h = jnp.dot(h, w2_ref[...], preferred_element_type=F32) + b2_ref[...]
    out_ref[...] = x_ref[...] + _ln(h, g_ref[...], be_ref[...])


def _dec_body(x_ref, w1_ref, b1_ref, w2_ref, b2_ref, out_ref):
    h = jnp.dot(x_ref[...], w1_ref[...], preferred_element_type=F32) + b1_ref[...]
    h = jnp.maximum(h, 0.0)
    out_ref[...] = jnp.dot(h, w2_ref[...], preferred_element_type=F32) + b2_ref[...]


def _full(a):
    nd = a.ndim
    return pl.BlockSpec(a.shape, lambda i, _nd=nd: (0,) * _nd)


def _pad_idx(idx, k):
    ng = idx.shape[0] // (k * LANES)
    idx3 = idx.reshape(ng, k, LANES)
    idx3 = jnp.pad(idx3, ((0, 0), (0, 8 - k), (0, 0)))
    return idx3.reshape(ng * 8, LANES)


@functools.lru_cache(maxsize=None)
def _make_gather(n_rows, n_tab):
    mesh = plsc.VectorSubcoreMesh(core_axis_name="c", subcore_axis_name="s")
    nw = mesh.num_cores * mesh.num_subcores
    ng = n_rows // GB
    k = GB // LANES

    @functools.partial(
        pl.kernel,
        out_type=jax.ShapeDtypeStruct((n_rows, H), F32),
        mesh=mesh,
        scratch_types=[
            pltpu.VMEM((8, LANES), jnp.int32),
            pltpu.VMEM((GB, H), F32),
            pltpu.SemaphoreType.DMA,
        ],
    )
    def gather_k(table, idx2d, out, idx_v, rows_v, sem):
        c = lax.axis_index("c")
        s = lax.axis_index("s")
        w = s * mesh.num_cores + c
        cnt = (ng - w + nw - 1) // nw

        @pl.loop(0, cnt)
        def _(t):
            g = w + t * nw
            pltpu.sync_copy(idx2d.at[pl.ds(g * 8, 8)], idx_v)
            descs = [
                pltpu.async_copy(table.at[idx_v.at[j]],
                                 rows_v.at[pl.ds(j * LANES, LANES)], sem)
                for j in range(k)
            ]
            for d in descs:
                d.wait()
            pltpu.sync_copy(rows_v, out.at[pl.ds(g * GB, GB)])

    return gather_k


@functools.lru_cache(maxsize=None)
def _make_scatter(n_edges, n_nodes):
    mesh = plsc.VectorSubcoreMesh(core_axis_name="c", subcore_axis_name="s")
    nc, ns = mesh.num_cores, mesh.num_subcores
    nw = nc * ns
    gb = 256
    ng = n_edges // gb
    k = gb // LANES
    assert n_nodes % 8 == 0
    nz = (n_nodes // ns) & ~7
    tail = n_nodes - ns * nz
    assert tail % 8 == 0 and tail <= gb
    zfull, zrem = nz // gb, nz % gb

    @functools.partial(
        pl.kernel,
        out_type=jax.ShapeDtypeStruct((nc, n_nodes, H), F32),
        mesh=mesh,
        scratch_types=[
            pltpu.VMEM((8, LANES), jnp.int32),
            pltpu.VMEM((gb, H), F32),
            pltpu.VMEM_SHARED((n_nodes, H), F32),
            pltpu.SemaphoreType.DMA,
        ],
    )
    def scatter_k(ea, col2d, out, idx_v, rows_v, acc, sem):
        c = lax.axis_index("c")
        s = lax.axis_index("s")
        w = s * nc + c
        base = s * nz

        @pl.loop(0, gb)
        def _(i):
            for j in range(H // 16):
                rows_v[i, pl.ds(j * 16, 16)] = jnp.zeros((16,), F32)

        for q in range(zfull):
            pltpu.sync_copy(rows_v, acc.at[pl.ds(base + q * gb, gb)])
        if zrem:
            pltpu.sync_copy(rows_v.at[pl.ds(0, zrem)],
                            acc.at[pl.ds(base + zfull * gb, zrem)])
        if tail:
            @pl.when(s == 0)
            def _():
                pltpu.sync_copy(rows_v.at[pl.ds(0, tail)],
                                acc.at[pl.ds(ns * nz, tail)])
        plsc.subcore_barrier()

        cnt = (ng - w + nw - 1) // nw

        @pl.loop(0, cnt)
        def _(t):
            g = w + t * nw
            pltpu.sync_copy(col2d.at[pl.ds(g * 8, 8)], idx_v)
            pltpu.sync_copy(ea.at[pl.ds(g * gb, gb)], rows_v)
            for j in range(k):
                pltpu.sync_copy(rows_v.at[pl.ds(j * LANES, LANES)],
                                acc.at[idx_v.at[j]], add=True)

        plsc.subcore_barrier()
        pltpu.sync_copy(acc.at[pl.ds(base, nz)], out.at[c, pl.ds(base, nz)])
        if tail:
            @pl.when(s == 0)
            def _():
                pltpu.sync_copy(acc.at[pl.ds(ns * nz, tail)],
                                out.at[c, pl.ds(ns * nz, tail)])

    return scatter_k


def kernel(x, edge_index, edge_attr, mean_vec_x, std_vec_x, mean_vec_edge,
           std_vec_edge, params):
    n, dn = x.shape
    e, de = edge_attr.shape
    assert e % GB == 0 and e % LANES == 0
    nblk = -(-n // TILE)
    npad = nblk * TILE
    eblk = e // TILE_E
    assert e % TILE_E == 0

    row = edge_index[0]
    col = edge_index[1]
    idx_all = _pad_idx(jnp.concatenate([row, col + npad]), GB // LANES)
    col2d = _pad_idx(col, 256 // LANES)

    r1 = lambda v: v.reshape(1, -1)
    par = pltpu.CompilerParams(dimension_semantics=("parallel",))
    par2 = pltpu.CompilerParams(dimension_semantics=("parallel", "parallel"))

    pe = params["node_enc"]
    x_enc = pl.pallas_call(
        _enc_node_body,
        grid=(nblk,),
        in_specs=[pl.BlockSpec((TILE, dn), lambda i: (i, 0))]
        + [_full(r1(mean_vec_x)), _full(r1(std_vec_x)), _full(pe["W1"]),
           _full(r1(pe["b1"])), _full(pe["W2"]), _full(r1(pe["b2"])),
           _full(r1(pe["g"])), _full(r1(pe["beta"]))],
        out_specs=pl.BlockSpec((TILE, H), lambda i: (i, 0)),
        out_shape=jax.ShapeDtypeStruct((n, H), F32),
        compiler_params=par,
    )(x, r1(mean_vec_x), r1(std_vec_x), pe["W1"], r1(pe["b1"]), pe["W2"],
      r1(pe["b2"]), r1(pe["g"]), r1(pe["beta"]))

    def _pq(xh, w1, b1):
        wpq = jnp.stack([w1[:H], w1[H:2 * H]])
        bpq = jnp.stack([jnp.zeros_like(b1), b1]).reshape(2, 1, H)
        return pl.pallas_call(
            _pq_body,
            grid=(2, nblk),
            in_specs=[pl.BlockSpec((TILE, H), lambda j, i: (i, 0)),
                      pl.BlockSpec((1, H, H), lambda j, i: (j, 0, 0)),
                      pl.BlockSpec((1, 1, H), lambda j, i: (j, 0, 0))],
            out_specs=pl.BlockSpec((1, TILE, H), lambda j, i: (j, i, 0)),
            out_shape=jax.ShapeDtypeStruct((2, npad, H), F32),
            compiler_params=par2,
        )(xh, wpq, bpq)

    gather = _make_gather(2 * e, 2 * npad)
    scatter = _make_scatter(e, n)

    ew = params["edge_enc"]
    xh = x_enc
    ea = None
    for li, lp in enumerate(params["layers"]):
        le, ln_ = lp["edge"], lp["node"]
        t = _pq(xh, le["W1"], le["b1"])
        s2 = gather(t.reshape(2 * npad, H), idx_all)

        espec = pl.BlockSpec((TILE_E, H), lambda i: (i, 0))
        gr_spec = pl.BlockSpec((TILE_E, H), lambda i: (i, 0))
        gc_spec = pl.BlockSpec((TILE_E, H), lambda i, _o=eblk: (i + _o, 0))
        w1e = le["W1"][2 * H:]
        if li == 0:
            ea = pl.pallas_call(
                _edge_first_body,
                grid=(eblk,),
                in_specs=[pl.BlockSpec((de, TILE_E), lambda i: (0, i)),
                          _full(mean_vec_edge.reshape(de, 1)),
                          _full(std_vec_edge.reshape(de, 1)),
                          _full(ew["W1"]), _full(r1(ew["b1"])), _full(ew["W2"]),
                          _full(r1(ew["b2"])), _full(r1(ew["g"])),
                          _full(r1(ew["beta"])), gr_spec, gc_spec,
                          _full(w1e), _full(le["W2"]), _full(r1(le["b2"])),
                          _full(r1(le["g"])), _full(r1(le["beta"]))],
                out_specs=pl.BlockSpec((TILE_E, H), lambda i: (i, 0)),
                out_shape=jax.ShapeDtypeStruct((e, H), F32),
                compiler_params=par,
            )(edge_attr.T, mean_vec_edge.reshape(de, 1),
              std_vec_edge.reshape(de, 1), ew["W1"],
              r1(ew["b1"]), ew["W2"], r1(ew["b2"]), r1(ew["g"]), r1(ew["beta"]),
              s2, s2, w1e, le["W2"], r1(le["b2"]), r1(le["g"]), r1(le["beta"]))
        else:
            ea = pl.pallas_call(
                _edge_rest_body,
                grid=(eblk,),
                in_specs=[espec, gr_spec, gc_spec, _full(w1e), _full(le["W2"]),
                          _full(r1(le["b2"])), _full(r1(le["g"])),
                          _full(r1(le["beta"]))],
                out_specs=pl.BlockSpec((TILE_E, H), lambda i: (i, 0)),
                out_shape=jax.ShapeDtypeStruct((e, H), F32),
                compiler_params=par,
            )(ea, s2, s2, w1e, le["W2"], r1(le["b2"]), r1(le["g"]),
              r1(le["beta"]))

        partials = scatter(ea, col2d)

        xh = pl.pallas_call(
            _node_body,
            grid=(nblk,),
            in_specs=[pl.BlockSpec((TILE, H), lambda i: (i, 0)),
                      pl.BlockSpec((1, TILE, H), lambda i: (0, i, 0)),
                      pl.BlockSpec((1, TILE, H), lambda i: (1, i, 0)),
                      _full(ln_["W1"][:H]), _full(ln_["W1"][H:]),
                      _full(r1(ln_["b1"])), _full(ln_["W2"]),
                      _full(r1(ln_["b2"])), _full(r1(ln_["g"])),
                      _full(r1(ln_["beta"]))],
            out_specs=pl.BlockSpec((TILE, H), lambda i: (i, 0)),
            out_shape=jax.ShapeDtypeStruct((n, H), F32),
            compiler_params=par,
        )(xh, partials, partials, ln_["W1"][:H], ln_["W1"][H:], r1(ln_["b1"]),
          ln_["W2"], r1(ln_["b2"]), r1(ln_["g"]), r1(ln_["beta"]))

    do = params["dec_W2"].shape[1]
    w2p = jnp.zeros((H, H), F32).at[:, :do].set(params["dec_W2"])
    b2p = jnp.zeros((1, H), F32).at[0, :do].set(params["dec_b2"])
    out = pl.pallas_call(
        _dec_body,
        grid=(nblk,),
        in_specs=[pl.BlockSpec((TILE, H), lambda i: (i, 0)),
                  _full(params["dec_W1"]), _full(r1(params["dec_b1"])),
                  _full(w2p), _full(b2p)],
        out_specs=pl.BlockSpec((TILE, H), lambda i: (i, 0)),
        out_shape=jax.ShapeDtypeStruct((n, H), F32),
        compiler_params=par,
    )(xh, params["dec_W1"], r1(params["dec_b1"]), w2p, b2p)
    return out[:, :do]


# speedup vs baseline: 1.7719x; 1.1854x over previous
import functools

import jax
import jax.numpy as jnp
from jax import lax
from jax.experimental import pallas as pl
from jax.experimental.pallas import tpu as pltpu
from jax.experimental.pallas import tpu_sc as plsc

H = 128
TILE = 512
TILE_E = 3200
GB = 512
LANES = 128

F32 = jnp.float32


def _ln(h, g, beta):
    mu = jnp.mean(h, axis=-1, keepdims=True)
    d = h - mu
    var = jnp.mean(d * d, axis=-1, keepdims=True)
    return d * lax.rsqrt(var + 1e-5) * g + beta


def _enc_node_body(x_ref, m_ref, s_ref, w1_ref, b1_ref, w2_ref, b2_ref,
                   g_ref, be_ref, out_ref):
    xn = (x_ref[...] - m_ref[...]) / s_ref[...]
    h = jnp.dot(xn, w1_ref[...], preferred_element_type=F32) + b1_ref[...]
    h = jnp.maximum(h, 0.0)
    h = jnp.dot(h, w2_ref[...], preferred_element_type=F32) + b2_ref[...]
    out_ref[...] = _ln(h, g_ref[...], be_ref[...])


def _pq_body(x_ref, w_ref, b_ref, out_ref):
    out_ref[0] = jnp.dot(x_ref[...], w_ref[0], preferred_element_type=F32) + b_ref[0]


BF16 = jnp.bfloat16


def _bdot(a, b):
    return jnp.dot(a.astype(BF16), b.astype(BF16), preferred_element_type=F32)


def _edge_first_body(er_ref, m_ref, s_ref, ew1_ref, eb1_ref, ew2_ref, eb2_ref,
                     eg_ref, ebe_ref, gr_ref, gc_ref, w1e_ref, w2_ref, b2_ref,
                     g_ref, be_ref, out_ref):
    ea = (er_ref[...] - m_ref[...]) / s_ref[...]
    h = lax.dot_general(ea, ew1_ref[...], (((0,), (0,)), ((), ())),
                        preferred_element_type=F32) + eb1_ref[...]
    h = jnp.maximum(h, 0.0)
    h = _bdot(h, ew2_ref[...]) + eb2_ref[...]
    ea_enc = _ln(h, eg_ref[...], ebe_ref[...])
    h1 = gr_ref[...] + gc_ref[...] + _bdot(ea_enc, w1e_ref[...])
    r = jnp.maximum(h1, 0.0)
    h2 = _bdot(r, w2_ref[...]) + b2_ref[...]
    out_ref[...] = ea_enc + _ln(h2, g_ref[...], be_ref[...])


def _edge_rest_body(ea_ref, gr_ref, gc_ref, w1e_ref, w2_ref, b2_ref,
                    g_ref, be_ref, out_ref):
    ea = ea_ref[...]
    h1 = gr_ref[...] + gc_ref[...] + _bdot(ea, w1e_ref[...])
    r = jnp.maximum(h1, 0.0)
    h2 = _bdot(r, w2_ref[...]) + b2_ref[...]
    out_ref[...] = ea + _ln(h2, g_ref[...], be_ref[...])


def _node_body_n(nparts):
    def body(*refs):
        x_ref = refs[0]
        parts = refs[1:1 + nparts]
        w1a_ref, w1b_ref, b1_ref, w2_ref, b2_ref, g_ref, be_ref, out_ref = \
            refs[1 + nparts:]
        agg = parts[0][0]
        for p_ref in parts[1:]:
            agg = agg + p_ref[0]
        h = (jnp.dot(x_ref[...], w1a_ref[...], preferred_element_type=F32)
             + jnp.dot(agg, w1b_ref[...], preferred_element_type=F32)
             + b1_ref[...])
        h = jnp.maximum(h, 0.0)
        h = jnp.dot(h, w2_ref[...], preferred_element_type=F32) + b2_ref[...]
        out_ref[...] = x_ref[...] + _ln(h, g_ref[...], be_ref[...])
    return body


def _dec_body(x_ref, w1_ref, b1_ref, w2_ref, b2_ref, out_ref):
    h = jnp.dot(x_ref[...], w1_ref[...], preferred_element_type=F32) + b1_ref[...]
    h = jnp.maximum(h, 0.0)
    out_ref[...] = jnp.dot(h, w2_ref[...], preferred_element_type=F32) + b2_ref[...]


def _full(a):
    nd = a.ndim
    return pl.BlockSpec(a.shape, lambda i, _nd=nd: (0,) * _nd)


def _pad_idx(idx, k):
    ng = idx.shape[0] // (k * LANES)
    idx3 = idx.reshape(ng, k, LANES)
    idx3 = jnp.pad(idx3, ((0, 0), (0, 8 - k), (0, 0)))
    return idx3.reshape(ng * 8, LANES)


@functools.lru_cache(maxsize=None)
def _make_gather(n_rows, n_tab):
    mesh = plsc.VectorSubcoreMesh(core_axis_name="c", subcore_axis_name="s")
    nw = mesh.num_cores * mesh.num_subcores
    ng = n_rows // GB
    k = GB // LANES

    @functools.partial(
        pl.kernel,
        out_type=jax.ShapeDtypeStruct((n_rows, H), F32),
        mesh=mesh,
        scratch_types=[
            pltpu.VMEM((8, LANES), jnp.int32),
            pltpu.VMEM((GB, H), F32),
            pltpu.SemaphoreType.DMA,
        ],
    )
    def gather_k(table, idx2d, out, idx_v, rows_v, sem):
        c = lax.axis_index("c")
        s = lax.axis_index("s")
        w = s * mesh.num_cores + c
        cnt = (ng - w + nw - 1) // nw

        @pl.loop(0, cnt)
        def _(t):
            g = w + t * nw
            pltpu.sync_copy(idx2d.at[pl.ds(g * 8, 8)], idx_v)
            descs = [
                pltpu.async_copy(table.at[idx_v.at[j]],
                                 rows_v.at[pl.ds(j * LANES, LANES)], sem)
                for j in range(k)
            ]
            for d in descs:
                d.wait()
            pltpu.sync_copy(rows_v, out.at[pl.ds(g * GB, GB)])

    return gather_k


@functools.lru_cache(maxsize=None)
def _make_scatter(n_edges, n_nodes):
    mesh = plsc.VectorSubcoreMesh(core_axis_name="c", subcore_axis_name="s")
    nc, ns = mesh.num_cores, mesh.num_subcores
    nw = nc * ns
    gb = 256
    ng = n_edges // gb
    k = gb // LANES
    assert n_nodes % 8 == 0
    nz = (n_nodes // ns) & ~7
    tail = n_nodes - ns * nz
    assert tail % 8 == 0 and tail <= gb
    zfull, zrem = nz // gb, nz % gb

    @functools.partial(
        pl.kernel,
        out_type=jax.ShapeDtypeStruct((nc, n_nodes, H), F32),
        mesh=mesh,
        scratch_types=[
            pltpu.VMEM((8, LANES), jnp.int32),
            pltpu.VMEM((gb, H), F32),
            pltpu.VMEM_SHARED((n_nodes, H), F32),
            pltpu.SemaphoreType.DMA,
        ],
    )
    def scatter_k(ea, col2d, out, idx_v, rows_v, acc, sem):
        c = lax.axis_index("c")
        s = lax.axis_index("s")
        w = s * nc + c
        base = s * nz

        @pl.loop(0, gb)
        def _(i):
            for j in range(H // 16):
                rows_v[i, pl.ds(j * 16, 16)] = jnp.zeros((16,), F32)

        for q in range(zfull):
            pltpu.sync_copy(rows_v, acc.at[pl.ds(base + q * gb, gb)])
        if zrem:
            pltpu.sync_copy(rows_v.at[pl.ds(0, zrem)],
                            acc.at[pl.ds(base + zfull * gb, zrem)])
        if tail:
            @pl.when(s == 0)
            def _():
                pltpu.sync_copy(rows_v.at[pl.ds(0, tail)],
                                acc.at[pl.ds(ns * nz, tail)])
        plsc.subcore_barrier()

        cnt = (ng - w + nw - 1) // nw

        @pl.loop(0, cnt)
        def _(t):
            g = w + t * nw
            pltpu.sync_copy(col2d.at[pl.ds(g * 8, 8)], idx_v)
            pltpu.sync_copy(ea.at[pl.ds(g * gb, gb)], rows_v)
            for j in range(k):
                pltpu.sync_copy(rows_v.at[pl.ds(j * LANES, LANES)],
                                acc.at[idx_v.at[j]], add=True)

        plsc.subcore_barrier()
        pltpu.sync_copy(acc.at[pl.ds(base, nz)], out.at[c, pl.ds(base, nz)])
        if tail:
            @pl.when(s == 0)
            def _():
                pltpu.sync_copy(acc.at[pl.ds(ns * nz, tail)],
                                out.at[c, pl.ds(ns * nz, tail)])

    return scatter_k


def kernel(x, edge_index, edge_attr, mean_vec_x, std_vec_x, mean_vec_edge,
           std_vec_edge, params):
    n, dn = x.shape
    e, de = edge_attr.shape
    assert e % GB == 0 and e % LANES == 0
    nblk = -(-n // TILE)
    npad = nblk * TILE
    eblk = e // TILE_E
    assert e % TILE_E == 0

    row = edge_index[0]
    col = edge_index[1]
    idx_all = _pad_idx(jnp.concatenate([row, col + npad]), GB // LANES)
    col2d = _pad_idx(col, 256 // LANES)

    r1 = lambda v: v.reshape(1, -1)
    par = pltpu.CompilerParams(dimension_semantics=("parallel",))
    par2 = pltpu.CompilerParams(dimension_semantics=("parallel", "parallel"))

    pe = params["node_enc"]
    x_enc = pl.pallas_call(
        _enc_node_body,
        grid=(nblk,),
        in_specs=[pl.BlockSpec((TILE, dn), lambda i: (i, 0))]
        + [_full(r1(mean_vec_x)), _full(r1(std_vec_x)), _full(pe["W1"]),
           _full(r1(pe["b1"])), _full(pe["W2"]), _full(r1(pe["b2"])),
           _full(r1(pe["g"])), _full(r1(pe["beta"]))],
        out_specs=pl.BlockSpec((TILE, H), lambda i: (i, 0)),
        out_shape=jax.ShapeDtypeStruct((n, H), F32),
        compiler_params=par,
    )(x, r1(mean_vec_x), r1(std_vec_x), pe["W1"], r1(pe["b1"]), pe["W2"],
      r1(pe["b2"]), r1(pe["g"]), r1(pe["beta"]))

    def _pq(xh, w1, b1):
        wpq = jnp.stack([w1[:H], w1[H:2 * H]])
        bpq = jnp.stack([jnp.zeros_like(b1), b1]).reshape(2, 1, H)
        return pl.pallas_call(
            _pq_body,
            grid=(2, nblk),
            in_specs=[pl.BlockSpec((TILE, H), lambda j, i: (i, 0)),
                      pl.BlockSpec((1, H, H), lambda j, i: (j, 0, 0)),
                      pl.BlockSpec((1, 1, H), lambda j, i: (j, 0, 0))],
            out_specs=pl.BlockSpec((1, TILE, H), lambda j, i: (j, i, 0)),
            out_shape=jax.ShapeDtypeStruct((2, npad, H), F32),
            compiler_params=par2,
        )(xh, wpq, bpq)

    nch = 2
    ech = e // nch
    assert ech % TILE_E == 0 and (2 * ech) % GB == 0 and ech % 256 == 0
    ecb = ech // TILE_E

    gather = _make_gather(2 * ech, 2 * npad)
    scatter = _make_scatter(ech, n)

    idx_ch, col_ch = [], []
    for ci in range(nch):
        r_c = row[ci * ech:(ci + 1) * ech]
        c_c = col[ci * ech:(ci + 1) * ech]
        idx_ch.append(_pad_idx(jnp.concatenate([r_c, c_c + npad]), GB // LANES))
        col_ch.append(_pad_idx(c_c, 256 // LANES))

    ew = params["edge_enc"]
    xh = x_enc
    ea = [None] * nch
    node_body = _node_body_n(2 * nch)
    gr_spec = pl.BlockSpec((TILE_E, H), lambda i: (i, 0))
    gc_spec = pl.BlockSpec((TILE_E, H), lambda i, _o=ecb: (i + _o, 0))
    for li, lp in enumerate(params["layers"]):
        le, ln_ = lp["edge"], lp["node"]
        t = _pq(xh, le["W1"], le["b1"])
        tflat = t.reshape(2 * npad, H)
        s2 = [gather(tflat, idx_ch[ci]) for ci in range(nch)]

        w1e = le["W1"][2 * H:]
        parts = []
        for ci in range(nch):
            if li == 0:
                ea[ci] = pl.pallas_call(
                    _edge_first_body,
                    grid=(ecb,),
                    in_specs=[pl.BlockSpec((de, TILE_E),
                                           lambda i, _c=ci * ecb: (0, i + _c)),
                              _full(mean_vec_edge.reshape(de, 1)),
                              _full(std_vec_edge.reshape(de, 1)),
                              _full(ew["W1"]), _full(r1(ew["b1"])),
                              _full(ew["W2"]), _full(r1(ew["b2"])),
                              _full(r1(ew["g"])), _full(r1(ew["beta"])),
                              gr_spec, gc_spec,
                              _full(w1e), _full(le["W2"]), _full(r1(le["b2"])),
                              _full(r1(le["g"])), _full(r1(le["beta"]))],
                    out_specs=pl.BlockSpec((TILE_E, H), lambda i: (i, 0)),
                    out_shape=jax.ShapeDtypeStruct((ech, H), F32),
                    compiler_params=par,
                )(edge_attr.T, mean_vec_edge.reshape(de, 1),
                  std_vec_edge.reshape(de, 1), ew["W1"],
                  r1(ew["b1"]), ew["W2"], r1(ew["b2"]), r1(ew["g"]),
                  r1(ew["beta"]), s2[ci], s2[ci], w1e, le["W2"],
                  r1(le["b2"]), r1(le["g"]), r1(le["beta"]))
            else:
                ea[ci] = pl.pallas_call(
                    _edge_rest_body,
                    grid=(ecb,),
                    in_specs=[pl.BlockSpec((TILE_E, H), lambda i: (i, 0)),
                              gr_spec, gc_spec, _full(w1e), _full(le["W2"]),
                              _full(r1(le["b2"])), _full(r1(le["g"])),
                              _full(r1(le["beta"]))],
                    out_specs=pl.BlockSpec((TILE_E, H), lambda i: (i, 0)),
                    out_shape=jax.ShapeDtypeStruct((ech, H), F32),
                    compiler_params=par,
                )(ea[ci], s2[ci], s2[ci], w1e, le["W2"], r1(le["b2"]),
                  r1(le["g"]), r1(le["beta"]))
            parts.append(scatter(ea[ci], col_ch[ci]))

        pspecs, pargs = [], []
        for pp in parts:
            pspecs += [pl.BlockSpec((1, TILE, H), lambda i: (0, i, 0)),
                       pl.BlockSpec((1, TILE, H), lambda i: (1, i, 0))]
            pargs += [pp, pp]
        xh = pl.pallas_call(
            node_body,
            grid=(nblk,),
            in_specs=[pl.BlockSpec((TILE, H), lambda i: (i, 0))] + pspecs
            + [_full(ln_["W1"][:H]), _full(ln_["W1"][H:]),
               _full(r1(ln_["b1"])), _full(ln_["W2"]),
               _full(r1(ln_["b2"])), _full(r1(ln_["g"])),
               _full(r1(ln_["beta"]))],
            out_specs=pl.BlockSpec((TILE, H), lambda i: (i, 0)),
            out_shape=jax.ShapeDtypeStruct((n, H), F32),
            compiler_params=par,
        )(xh, *pargs, ln_["W1"][:H], ln_["W1"][H:], r1(ln_["b1"]),
          ln_["W2"], r1(ln_["b2"]), r1(ln_["g"]), r1(ln_["beta"]))

    do = params["dec_W2"].shape[1]
    w2p = jnp.zeros((H, H), F32).at[:, :do].set(params["dec_W2"])
    b2p = jnp.zeros((1, H), F32).at[0, :do].set(params["dec_b2"])
    out = pl.pallas_call(
        _dec_body,
        grid=(nblk,),
        in_specs=[pl.BlockSpec((TILE, H), lambda i: (i, 0)),
                  _full(params["dec_W1"]), _full(r1(params["dec_b1"])),
                  _full(w2p), _full(b2p)],
        out_specs=pl.BlockSpec((TILE, H), lambda i: (i, 0)),
        out_shape=jax.ShapeDtypeStruct((n, H), F32),
        compiler_params=par,
    )(xh, params["dec_W1"], r1(params["dec_b1"]), w2p, b2p)
    return out[:, :do]
